# Initial kernel scaffold; baseline (speedup 1.0000x reference)
import jax
import jax.numpy as jnp
from jax.experimental import pallas as pl


def kernel(x):
    raise NotImplementedError("write your pallas kernel here")


# speedup vs baseline: 5.2168x; 5.2168x over previous
import jax
import jax.numpy as jnp
from jax.experimental import pallas as pl
from jax.experimental.pallas import tpu as pltpu

_B, _C, _H, _W = 64, 128, 112, 112
_OH, _OW = 56, 56
_N = _B * _C * _OH
_G = 1024


def _quant(v):
    return v.astype(jnp.bfloat16).astype(jnp.float32)


def _pool_body(x_ref, o_ref):
    x = x_ref[...]
    r0 = x[:, :_W]
    r1 = x[:, _W:]
    even = 2 * jax.lax.broadcasted_iota(jnp.int32, (_G, _OW), 1)
    odd = even + 1
    a = jnp.take_along_axis(r0, even, axis=1)
    b = jnp.take_along_axis(r0, odd, axis=1)
    c = jnp.take_along_axis(r1, even, axis=1)
    d = jnp.take_along_axis(r1, odd, axis=1)
    y = _quant(a)
    y = _quant(y + b)
    y = _quant(y + c)
    y = _quant(y + d)
    o_ref[...] = _quant(y * 0.25)


def kernel(x):
    xf = x.reshape(_N, 2 * _W)
    out = pl.pallas_call(
        _pool_body,
        grid=(_N // _G,),
        in_specs=[pl.BlockSpec((_G, 2 * _W), lambda i: (i, 0))],
        out_specs=pl.BlockSpec((_G, _OW), lambda i: (i, 0)),
        out_shape=jax.ShapeDtypeStruct((_N, _OW), jnp.float32),
        compiler_params=pltpu.CompilerParams(
            dimension_semantics=("parallel",),
        ),
    )(xf)
    return out.reshape(_B, _C, _OH, _OW)


# speedup vs baseline: 9.4396x; 1.8094x over previous
import jax
import jax.numpy as jnp
from jax.experimental import pallas as pl
from jax.experimental.pallas import tpu as pltpu

_B, _C, _H, _W = 64, 128, 112, 112
_OH, _OW = 56, 56
_BC = _B * _C
_G = 64
_T = _H // 16


def _quant(v):
    return v.astype(jnp.bfloat16).astype(jnp.float32)


def _pool_body(x_ref, o_ref):
    x = x_ref[...].reshape(_G, _T, 2, 8, _W)
    lane_even = 2 * jax.lax.broadcasted_iota(jnp.int32, (_G, _T, 2, 8, _OW), 4)
    a = jnp.take_along_axis(x, lane_even, axis=4)
    b = jnp.take_along_axis(x, lane_even + 1, axis=4)
    u = _quant(_quant(a) + b)
    si = jax.lax.broadcasted_iota(jnp.int32, (_G, _T, 2, 8, _OW), 3)
    idx_e = 2 * (si % 4)
    ue = jnp.take_along_axis(u, idx_e, axis=3)
    ao = jnp.take_along_axis(a, idx_e + 1, axis=3)
    bo = jnp.take_along_axis(b, idx_e + 1, axis=3)
    y = _quant(ue + ao)
    y = _quant(y + bo)
    res = _quant(y * 0.25)
    si4 = jax.lax.broadcasted_iota(jnp.int32, (_G, _T, 8, _OW), 2)
    out = jnp.where(si4 < 4, res[:, :, 0, :, :], res[:, :, 1, :, :])
    o_ref[...] = out.reshape(_G, _OH, _OW)


def kernel(x):
    xf = x.reshape(_BC, _H, _W)
    out = pl.pallas_call(
        _pool_body,
        grid=(_BC // _G,),
        in_specs=[pl.BlockSpec((_G, _H, _W), lambda i: (i, 0, 0))],
        out_specs=pl.BlockSpec((_G, _OH, _OW), lambda i: (i, 0, 0)),
        out_shape=jax.ShapeDtypeStruct((_BC, _OH, _OW), jnp.float32),
        compiler_params=pltpu.CompilerParams(
            dimension_semantics=("parallel",),
        ),
    )(xf)
    return out.reshape(_B, _C, _OH, _OW)
